# baseline (speedup 1.0000x reference)
import functools

import jax
import jax.numpy as jnp
from jax import lax
from jax.experimental import pallas as pl
from jax.experimental.pallas import tpu as pltpu
from jax.experimental.pallas import tpu_sc as plsc

_N = 10000
_E = 320000
_D = 128
_NC = 2
_NS = 16
_NW = _NC * _NS
_L = 16
_DG = 2 * _D
_CH = 80
_EPW = _E // _NW
_NCH = _EPW // _CH
_CHS = 40
_NCHS = _EPW // _CHS
_RPT = 624
_BE = 2560
_SLICES = ((0, 40), (40, 40), (80, 30), (110, 15))
_F32 = jnp.float32


def _gn(x, g, b):
    m = jnp.mean(x, axis=1, keepdims=True)
    v = jnp.mean((x - m) ** 2, axis=1, keepdims=True)
    return (x - m) * lax.rsqrt(v + 1e-5) * g + b


def _nodes_tc(agts, ctrs, wd1, b1, wq, wc1q, wc1c, wagt, gq_g, gq_b):
    def body(x_ref, c_ref, wd1_ref, b1_ref, wq_ref, wc1q_ref, wc1c_ref,
             wagt_ref, g_ref, b_ref, a_ref, bt_ref, p_ref, pn_ref, o0_ref):
        x = x_ref[...]
        q = jnp.dot(x, wq_ref[...], preferred_element_type=_F32)
        q = jnp.maximum(_gn(q, g_ref[...], b_ref[...]), 0.0)
        a_ref[...] = jnp.dot(q, wc1q_ref[...], preferred_element_type=_F32)
        bt_ref[...] = jnp.dot(x, wc1c_ref[...], preferred_element_type=_F32)
        p = jnp.dot(c_ref[...], wd1_ref[...], preferred_element_type=_F32)
        p_ref[...] = p
        pn_ref[...] = b1_ref[...] - p
        o0_ref[...] = jnp.dot(x, wagt_ref[...], preferred_element_type=_F32)

    return pl.pallas_call(
        body,
        out_shape=[jax.ShapeDtypeStruct((_N, _D), _F32)] * 5,
    )(agts, ctrs, wd1, b1, wq, wc1q, wc1c, wagt, gq_g, gq_b)


def _gather_sc(hi2, wi2, ta, tb, tp, tn, c0, ncs):
    mesh = plsc.VectorSubcoreMesh(core_axis_name="c", subcore_axis_name="s")
    es = _NW * ncs * _CH

    @functools.partial(
        pl.kernel,
        out_type=(jax.ShapeDtypeStruct((es, _D), _F32),
                  jax.ShapeDtypeStruct((es, _D), _F32)),
        mesh=mesh,
        scratch_types=[
            pltpu.VMEM((_NCH, _CH), jnp.int32),
            pltpu.VMEM((_NCH, _CH), jnp.int32),
            pltpu.VMEM((_CH, _D), _F32),
            pltpu.VMEM((_CH, _D), _F32),
            pltpu.SemaphoreType.DMA,
            pltpu.SemaphoreType.DMA,
        ],
        compiler_params=pltpu.CompilerParams(use_tc_tiling_on_sc=False),
    )
    def k(hi_hbm, wi_hbm, ta_hbm, tb_hbm, tp_hbm, tn_hbm, g1_hbm, g0_hbm,
          hiv, wiv, rows1, rows0, sem1, sem0):
        c = lax.axis_index("c")
        s = lax.axis_index("s")
        w = c * _NS + s
        pltpu.sync_copy(hi_hbm.at[w], hiv)
        pltpu.sync_copy(wi_hbm.at[w], wiv)

        def chunk(j, carry):
            i = c0 + j
            off = w * (ncs * _CH) + j * _CH
            c1 = pltpu.async_copy(ta_hbm.at[hiv.at[i]], rows1, sem1)
            cc0 = pltpu.async_copy(tp_hbm.at[hiv.at[i]], rows0, sem0)
            c1.wait()
            c1 = pltpu.async_copy(tb_hbm.at[wiv.at[i]], rows1, sem1,
                                  add=True)
            cc0.wait()
            cc0 = pltpu.async_copy(tn_hbm.at[wiv.at[i]], rows0, sem0,
                                   add=True)
            c1.wait()
            pltpu.sync_copy(rows1, g1_hbm.at[pl.ds(off, _CH)])
            cc0.wait()
            pltpu.sync_copy(rows0, g0_hbm.at[pl.ds(off, _CH)])
            return carry

        lax.fori_loop(0, ncs, chunk, 0)

    return k(hi2, wi2, ta, tb, tp, tn)


def _edges_tc(g1, g0, wd2, gd_g, gd_b, wc1d, gc_g, gc_b):
    def body(g1_ref, g0_ref, wd2_ref, gdg_ref, gdb_ref,
             wc1d_ref, gcg_ref, gcb_ref, h_ref):
        m0 = jnp.maximum(g0_ref[...], 0.0)
        t = jnp.dot(m0.astype(jnp.bfloat16), wd2_ref[...].astype(jnp.bfloat16),
                    preferred_element_type=_F32)
        m1 = jnp.maximum(_gn(t, gdg_ref[...], gdb_ref[...]), 0.0)
        pre = jnp.dot(m1.astype(jnp.bfloat16),
                      wc1d_ref[...].astype(jnp.bfloat16),
                      preferred_element_type=_F32) \
            + g1_ref[...]
        h_ref[...] = jnp.maximum(_gn(pre, gcg_ref[...], gcb_ref[...]), 0.0)

    full = lambda shape: pl.BlockSpec(shape, lambda i: (0, 0))
    es = g1.shape[0]
    return pl.pallas_call(
        body,
        grid=(es // _BE,),
        in_specs=[
            pl.BlockSpec((_BE, _D), lambda i: (i, 0)),
            pl.BlockSpec((_BE, _D), lambda i: (i, 0)),
            full((_D, _D)),
            full((1, _D)), full((1, _D)), full((_D, _D)),
            full((1, _D)), full((1, _D)),
        ],
        out_specs=pl.BlockSpec((_BE, _D), lambda i: (i, 0)),
        out_shape=jax.ShapeDtypeStruct((es, _D), _F32),
    )(g1, g0, wd2, gd_g, gd_b, wc1d, gc_g, gc_b)


def _scatter_sc(hi3, h, zeros, c0s, ncs2):
    mesh = plsc.VectorSubcoreMesh(core_axis_name="c", subcore_axis_name="s")
    nc2 = ncs2 // 2

    @functools.partial(
        pl.kernel,
        out_type=jax.ShapeDtypeStruct((_NC, _N, _D), _F32),
        mesh=mesh,
        scratch_types=[
            pltpu.VMEM((_NCHS, _CHS), jnp.int32),
            pltpu.VMEM((_CHS, _D), _F32),
            pltpu.VMEM((_CHS, _D), _F32),
            pltpu.VMEM_SHARED((_N, _D), _F32),
            pltpu.SemaphoreType.DMA,
            pltpu.SemaphoreType.DMA,
        ],
        compiler_params=pltpu.CompilerParams(use_tc_tiling_on_sc=False),
    )
    def k(hi_hbm, h_hbm, z_hbm, out_hbm, hiv, rows0, rows1, acc,
          sem0, sem1):
        c = lax.axis_index("c")
        s = lax.axis_index("s")
        w = c * _NS + s
        base = w * (ncs2 * _CHS)
        pltpu.sync_copy(z_hbm.at[pl.ds(s * _RPT, _RPT)],
                        acc.at[pl.ds(s * _RPT, _RPT)])
        @pl.when(s == 0)
        def _():
            pltpu.sync_copy(z_hbm.at[pl.ds(_NS * _RPT, _N - _NS * _RPT)],
                            acc.at[pl.ds(_NS * _RPT, _N - _NS * _RPT)])
        pltpu.sync_copy(hi_hbm.at[w], hiv)
        plsc.subcore_barrier()

        def ld(i, rows, sem):
            return pltpu.async_copy(h_hbm.at[pl.ds(base + i * _CHS, _CHS)],
                                    rows, sem)

        def wait0():
            pltpu.make_async_copy(h_hbm.at[pl.ds(base, _CHS)], rows0,
                                  sem0).wait()

        def wait1():
            pltpu.make_async_copy(h_hbm.at[pl.ds(base, _CHS)], rows1,
                                  sem1).wait()

        ld(0, rows0, sem0)

        def pair(j, carry):
            i = 2 * j
            ld(i + 1, rows1, sem1)
            wait0()
            pltpu.sync_copy(rows0, acc.at[hiv.at[c0s + i]], add=True)
            ld(i + 2, rows0, sem0)
            wait1()
            pltpu.sync_copy(rows1, acc.at[hiv.at[c0s + i + 1]], add=True)
            return carry

        lax.fori_loop(0, nc2 - 1, pair, 0)
        i = 2 * (nc2 - 1)
        ld(i + 1, rows1, sem1)
        wait0()
        pltpu.sync_copy(rows0, acc.at[hiv.at[c0s + i]], add=True)
        wait1()
        pltpu.sync_copy(rows1, acc.at[hiv.at[c0s + i + 1]], add=True)

        plsc.subcore_barrier()
        pltpu.sync_copy(acc.at[pl.ds(s * _RPT, _RPT)],
                        out_hbm.at[c, pl.ds(s * _RPT, _RPT)])
        @pl.when(s == 0)
        def _():
            pltpu.sync_copy(acc.at[pl.ds(_NS * _RPT, _N - _NS * _RPT)],
                            out_hbm.at[c, pl.ds(_NS * _RPT, _N - _NS * _RPT)])

    return k(hi3, h, zeros)


def _epi_tc(out0, parts, agts, wc2, ga_g, ga_b, wlin, gl_g, gl_b):
    def body(o0_ref, p0_ref, p1_ref, p2_ref, p3_ref, p4_ref, p5_ref,
             p6_ref, p7_ref, a_ref, wc2_ref, gag_ref, gab_ref, wl_ref,
             glg_ref, glb_ref, out_ref):
        sacc = (((p0_ref[...] + p1_ref[...]) + (p2_ref[...] + p3_ref[...]))
                + ((p4_ref[...] + p5_ref[...]) + (p6_ref[...] + p7_ref[...])))
        u = o0_ref[...] + jnp.dot(sacc, wc2_ref[...],
                                  preferred_element_type=_F32)
        u = jnp.maximum(_gn(u, gag_ref[...], gab_ref[...]), 0.0)
        v = _gn(jnp.dot(u, wl_ref[...], preferred_element_type=_F32),
                glg_ref[...], glb_ref[...])
        out_ref[...] = jnp.maximum(v + a_ref[...], 0.0)

    rb = 2000
    row = pl.BlockSpec((rb, _D), lambda i: (i, 0))
    full = lambda shape: pl.BlockSpec(shape, lambda i: (0, 0))
    return pl.pallas_call(
        body,
        grid=(_N // rb,),
        in_specs=[row] * (1 + len(parts) + 1)
        + [full((_D, _D)), full((1, _D)), full((1, _D)),
           full((_D, _D)), full((1, _D)), full((1, _D))],
        out_specs=row,
        out_shape=jax.ShapeDtypeStruct((_N, _D), _F32),
    )(out0, *parts, agts, wc2, ga_g, ga_b, wlin, gl_g, gl_b)


def kernel(agts, agt_ctrs, edge_index, W_dist1, b_dist1, W_dist2, gn_dist_g,
           gn_dist_b, W_query, gn_query_g, gn_query_b, W_ctx1, gn_ctx1_g,
           gn_ctx1_b, W_ctx2, W_agt, gn_agg_g, gn_agg_b, W_lin, gn_lin_g,
           gn_lin_b):
    r2 = lambda p: p.reshape(1, _D).astype(_F32)
    hi = edge_index[0].astype(jnp.int32)
    wi = edge_index[1].astype(jnp.int32)
    hi2 = hi.reshape(_NW, _NCH, _CH)
    wi2 = wi.reshape(_NW, _NCH, _CH)
    hi3 = hi.reshape(_NW, _NCHS, _CHS)

    a_tab, b_tab, p_tab, pn_tab, out0 = _nodes_tc(
        agts, agt_ctrs, W_dist1.T, b_dist1.reshape(1, _D), W_query.T,
        W_ctx1[:, _D:2 * _D].T, W_ctx1[:, 2 * _D:].T, W_agt.T,
        r2(gn_query_g), r2(gn_query_b))

    zeros = jnp.zeros((_N, _D), _F32)
    gs = [_gather_sc(hi2, wi2, a_tab, b_tab, p_tab, pn_tab, c0, ncs)
          for (c0, ncs) in _SLICES]
    parts = []
    for (c0, ncs), (g1, g0) in zip(_SLICES, gs):
        h_s = _edges_tc(
            g1, g0, W_dist2.T,
            r2(gn_dist_g), r2(gn_dist_b), W_ctx1[:, :_D].T,
            r2(gn_ctx1_g), r2(gn_ctx1_b))
        parts.append(_scatter_sc(hi3, h_s, zeros, c0 * 2, ncs * 2))

    flat = [p[i] for p in parts for i in range(_NC)]
    return _epi_tc(out0, flat, agts, W_ctx2.T, r2(gn_agg_g), r2(gn_agg_b),
                   W_lin.T, r2(gn_lin_g), r2(gn_lin_b))


# speedup vs baseline: 1.0443x; 1.0443x over previous
import functools

import jax
import jax.numpy as jnp
from jax import lax
from jax.experimental import pallas as pl
from jax.experimental.pallas import tpu as pltpu
from jax.experimental.pallas import tpu_sc as plsc

_N = 10000
_E = 320000
_D = 128
_NC = 2
_NS = 16
_NW = _NC * _NS
_L = 16
_DG = 2 * _D
_CH = 80
_EPW = _E // _NW
_NCH = _EPW // _CH
_CHS = 40
_NCHS = _EPW // _CHS
_RPT = 624
_BE = 2560
_SLICES = ((0, 50), (50, 50), (100, 25))
_F32 = jnp.float32


def _gn(x, g, b):
    m = jnp.mean(x, axis=1, keepdims=True)
    v = jnp.mean((x - m) ** 2, axis=1, keepdims=True)
    return (x - m) * lax.rsqrt(v + 1e-5) * g + b


def _nodes_tc(agts, ctrs, wd1, b1, wq, wc1q, wc1c, wagt, gq_g, gq_b):
    def body(x_ref, c_ref, wd1_ref, b1_ref, wq_ref, wc1q_ref, wc1c_ref,
             wagt_ref, g_ref, b_ref, a_ref, bt_ref, p_ref, pn_ref, o0_ref):
        x = x_ref[...]
        q = jnp.dot(x, wq_ref[...], preferred_element_type=_F32)
        q = jnp.maximum(_gn(q, g_ref[...], b_ref[...]), 0.0)
        a_ref[...] = jnp.dot(q, wc1q_ref[...], preferred_element_type=_F32)
        bt_ref[...] = jnp.dot(x, wc1c_ref[...], preferred_element_type=_F32)
        p = jnp.dot(c_ref[...], wd1_ref[...], preferred_element_type=_F32)
        p_ref[...] = p
        pn_ref[...] = b1_ref[...] - p
        o0_ref[...] = jnp.dot(x, wagt_ref[...], preferred_element_type=_F32)

    return pl.pallas_call(
        body,
        out_shape=[jax.ShapeDtypeStruct((_N, _D), _F32)] * 5,
    )(agts, ctrs, wd1, b1, wq, wc1q, wc1c, wagt, gq_g, gq_b)


def _gather_sc(hi2, wi2, ta, tb, tp, tn, c0, ncs):
    mesh = plsc.VectorSubcoreMesh(core_axis_name="c", subcore_axis_name="s")
    es = _NW * ncs * _CH

    @functools.partial(
        pl.kernel,
        out_type=(jax.ShapeDtypeStruct((es, _D), _F32),
                  jax.ShapeDtypeStruct((es, _D), _F32)),
        mesh=mesh,
        scratch_types=[
            pltpu.VMEM((_NCH, _CH), jnp.int32),
            pltpu.VMEM((_NCH, _CH), jnp.int32),
            pltpu.VMEM((_CH, _D), _F32),
            pltpu.VMEM((_CH, _D), _F32),
            pltpu.SemaphoreType.DMA,
            pltpu.SemaphoreType.DMA,
        ],
        compiler_params=pltpu.CompilerParams(use_tc_tiling_on_sc=False),
    )
    def k(hi_hbm, wi_hbm, ta_hbm, tb_hbm, tp_hbm, tn_hbm, g1_hbm, g0_hbm,
          hiv, wiv, rows1, rows0, sem1, sem0):
        c = lax.axis_index("c")
        s = lax.axis_index("s")
        w = c * _NS + s
        pltpu.sync_copy(hi_hbm.at[w], hiv)
        pltpu.sync_copy(wi_hbm.at[w], wiv)

        def chunk(j, carry):
            i = c0 + j
            off = w * (ncs * _CH) + j * _CH
            c1 = pltpu.async_copy(ta_hbm.at[hiv.at[i]], rows1, sem1)
            cc0 = pltpu.async_copy(tp_hbm.at[hiv.at[i]], rows0, sem0)
            c1.wait()
            c1 = pltpu.async_copy(tb_hbm.at[wiv.at[i]], rows1, sem1,
                                  add=True)
            cc0.wait()
            cc0 = pltpu.async_copy(tn_hbm.at[wiv.at[i]], rows0, sem0,
                                   add=True)
            c1.wait()
            pltpu.sync_copy(rows1, g1_hbm.at[pl.ds(off, _CH)])
            cc0.wait()
            pltpu.sync_copy(rows0, g0_hbm.at[pl.ds(off, _CH)])
            return carry

        lax.fori_loop(0, ncs, chunk, 0)

    return k(hi2, wi2, ta, tb, tp, tn)


def _edges_tc(g1, g0, wd2, gd_g, gd_b, wc1d, gc_g, gc_b):
    def body(g1_ref, g0_ref, wd2_ref, gdg_ref, gdb_ref,
             wc1d_ref, gcg_ref, gcb_ref, h_ref):
        m0 = jnp.maximum(g0_ref[...], 0.0)
        t = jnp.dot(m0.astype(jnp.bfloat16), wd2_ref[...].astype(jnp.bfloat16),
                    preferred_element_type=_F32)
        m1 = jnp.maximum(_gn(t, gdg_ref[...], gdb_ref[...]), 0.0)
        pre = jnp.dot(m1.astype(jnp.bfloat16),
                      wc1d_ref[...].astype(jnp.bfloat16),
                      preferred_element_type=_F32) \
            + g1_ref[...]
        h_ref[...] = jnp.maximum(_gn(pre, gcg_ref[...], gcb_ref[...]), 0.0)

    full = lambda shape: pl.BlockSpec(shape, lambda i: (0, 0))
    es = g1.shape[0]
    return pl.pallas_call(
        body,
        grid=(es // _BE,),
        in_specs=[
            pl.BlockSpec((_BE, _D), lambda i: (i, 0)),
            pl.BlockSpec((_BE, _D), lambda i: (i, 0)),
            full((_D, _D)),
            full((1, _D)), full((1, _D)), full((_D, _D)),
            full((1, _D)), full((1, _D)),
        ],
        out_specs=pl.BlockSpec((_BE, _D), lambda i: (i, 0)),
        out_shape=jax.ShapeDtypeStruct((es, _D), _F32),
    )(g1, g0, wd2, gd_g, gd_b, wc1d, gc_g, gc_b)


def _scatter_sc(hi3, h, zeros, c0s, ncs2):
    mesh = plsc.VectorSubcoreMesh(core_axis_name="c", subcore_axis_name="s")
    nc2 = ncs2 // 2

    @functools.partial(
        pl.kernel,
        out_type=jax.ShapeDtypeStruct((_NC, _N, _D), _F32),
        mesh=mesh,
        scratch_types=[
            pltpu.VMEM((_NCHS, _CHS), jnp.int32),
            pltpu.VMEM((_CHS, _D), _F32),
            pltpu.VMEM((_CHS, _D), _F32),
            pltpu.VMEM_SHARED((_N, _D), _F32),
            pltpu.SemaphoreType.DMA,
            pltpu.SemaphoreType.DMA,
        ],
        compiler_params=pltpu.CompilerParams(use_tc_tiling_on_sc=False),
    )
    def k(hi_hbm, h_hbm, z_hbm, out_hbm, hiv, rows0, rows1, acc,
          sem0, sem1):
        c = lax.axis_index("c")
        s = lax.axis_index("s")
        w = c * _NS + s
        base = w * (ncs2 * _CHS)
        pltpu.sync_copy(z_hbm.at[pl.ds(s * _RPT, _RPT)],
                        acc.at[pl.ds(s * _RPT, _RPT)])
        @pl.when(s == 0)
        def _():
            pltpu.sync_copy(z_hbm.at[pl.ds(_NS * _RPT, _N - _NS * _RPT)],
                            acc.at[pl.ds(_NS * _RPT, _N - _NS * _RPT)])
        pltpu.sync_copy(hi_hbm.at[w], hiv)
        plsc.subcore_barrier()

        def ld(i, rows, sem):
            return pltpu.async_copy(h_hbm.at[pl.ds(base + i * _CHS, _CHS)],
                                    rows, sem)

        def wait0():
            pltpu.make_async_copy(h_hbm.at[pl.ds(base, _CHS)], rows0,
                                  sem0).wait()

        def wait1():
            pltpu.make_async_copy(h_hbm.at[pl.ds(base, _CHS)], rows1,
                                  sem1).wait()

        ld(0, rows0, sem0)

        def pair(j, carry):
            i = 2 * j
            ld(i + 1, rows1, sem1)
            wait0()
            pltpu.sync_copy(rows0, acc.at[hiv.at[c0s + i]], add=True)
            ld(i + 2, rows0, sem0)
            wait1()
            pltpu.sync_copy(rows1, acc.at[hiv.at[c0s + i + 1]], add=True)
            return carry

        lax.fori_loop(0, nc2 - 1, pair, 0)
        i = 2 * (nc2 - 1)
        ld(i + 1, rows1, sem1)
        wait0()
        pltpu.sync_copy(rows0, acc.at[hiv.at[c0s + i]], add=True)
        wait1()
        pltpu.sync_copy(rows1, acc.at[hiv.at[c0s + i + 1]], add=True)

        plsc.subcore_barrier()
        pltpu.sync_copy(acc.at[pl.ds(s * _RPT, _RPT)],
                        out_hbm.at[c, pl.ds(s * _RPT, _RPT)])
        @pl.when(s == 0)
        def _():
            pltpu.sync_copy(acc.at[pl.ds(_NS * _RPT, _N - _NS * _RPT)],
                            out_hbm.at[c, pl.ds(_NS * _RPT, _N - _NS * _RPT)])

    return k(hi3, h, zeros)


def _epi_tc(out0, parts, agts, wc2, ga_g, ga_b, wlin, gl_g, gl_b):
    def body(o0_ref, p0_ref, p1_ref, p2_ref, p3_ref, p4_ref, p5_ref,
             a_ref, wc2_ref, gag_ref, gab_ref, wl_ref, glg_ref, glb_ref,
             out_ref):
        sacc = ((p0_ref[...] + p1_ref[...]) + (p2_ref[...] + p3_ref[...])
                + (p4_ref[...] + p5_ref[...]))
        u = o0_ref[...] + jnp.dot(sacc, wc2_ref[...],
                                  preferred_element_type=_F32)
        u = jnp.maximum(_gn(u, gag_ref[...], gab_ref[...]), 0.0)
        v = _gn(jnp.dot(u, wl_ref[...], preferred_element_type=_F32),
                glg_ref[...], glb_ref[...])
        out_ref[...] = jnp.maximum(v + a_ref[...], 0.0)

    rb = 2000
    row = pl.BlockSpec((rb, _D), lambda i: (i, 0))
    full = lambda shape: pl.BlockSpec(shape, lambda i: (0, 0))
    return pl.pallas_call(
        body,
        grid=(_N // rb,),
        in_specs=[row] * (1 + len(parts) + 1)
        + [full((_D, _D)), full((1, _D)), full((1, _D)),
           full((_D, _D)), full((1, _D)), full((1, _D))],
        out_specs=row,
        out_shape=jax.ShapeDtypeStruct((_N, _D), _F32),
    )(out0, *parts, agts, wc2, ga_g, ga_b, wlin, gl_g, gl_b)


def kernel(agts, agt_ctrs, edge_index, W_dist1, b_dist1, W_dist2, gn_dist_g,
           gn_dist_b, W_query, gn_query_g, gn_query_b, W_ctx1, gn_ctx1_g,
           gn_ctx1_b, W_ctx2, W_agt, gn_agg_g, gn_agg_b, W_lin, gn_lin_g,
           gn_lin_b):
    r2 = lambda p: p.reshape(1, _D).astype(_F32)
    hi = edge_index[0].astype(jnp.int32)
    wi = edge_index[1].astype(jnp.int32)
    hi2 = hi.reshape(_NW, _NCH, _CH)
    wi2 = wi.reshape(_NW, _NCH, _CH)
    hi3 = hi.reshape(_NW, _NCHS, _CHS)

    a_tab, b_tab, p_tab, pn_tab, out0 = _nodes_tc(
        agts, agt_ctrs, W_dist1.T, b_dist1.reshape(1, _D), W_query.T,
        W_ctx1[:, _D:2 * _D].T, W_ctx1[:, 2 * _D:].T, W_agt.T,
        r2(gn_query_g), r2(gn_query_b))

    zeros = jnp.zeros((_N, _D), _F32)
    gs = [_gather_sc(hi2, wi2, a_tab, b_tab, p_tab, pn_tab, c0, ncs)
          for (c0, ncs) in _SLICES]
    parts = []
    for (c0, ncs), (g1, g0) in zip(_SLICES, gs):
        h_s = _edges_tc(
            g1, g0, W_dist2.T,
            r2(gn_dist_g), r2(gn_dist_b), W_ctx1[:, :_D].T,
            r2(gn_ctx1_g), r2(gn_ctx1_b))
        parts.append(_scatter_sc(hi3, h_s, zeros, c0 * 2, ncs * 2))

    flat = [p[i] for p in parts for i in range(_NC)]
    return _epi_tc(out0, flat, agts, W_ctx2.T, r2(gn_agg_g), r2(gn_agg_b),
                   W_lin.T, r2(gn_lin_g), r2(gn_lin_b))
